# baseline (speedup 1.0000x reference)
import functools

import jax
import jax.numpy as jnp
from jax import lax
from jax.experimental import pallas as pl
from jax.experimental.pallas import tpu as pltpu
from jax.experimental.pallas import tpu_sc as plsc

N = 50000
E = 800000
NPAD = 51200
K = 6400
NCHUNK = NPAD // K
NSC = 2
NTILE = 16
EPT = E // NTILE
S = 2000
NSEG = EPT // S
SENT = S
RPT = K // NTILE
BR = 512
LANES = 16


def _dense(fin, totw):
    def body(x_ref, w_ref, wa_ref, h_ref, aux_ref):
        xb = x_ref[...]
        h_ref[...] = jnp.dot(xb, w_ref[...], preferred_element_type=jnp.float32)
        aux_ref[...] = jnp.dot(xb, wa_ref[...], preferred_element_type=jnp.float32)

    grid = NPAD // BR
    return pl.pallas_call(
        body,
        grid=(grid,),
        in_specs=[
            pl.BlockSpec((BR, fin), lambda i: (i, 0)),
            pl.BlockSpec((fin, totw), lambda i: (0, 0)),
            pl.BlockSpec((fin, 16), lambda i: (0, 0)),
        ],
        out_specs=[
            pl.BlockSpec((BR, totw), lambda i: (i, 0)),
            pl.BlockSpec((BR, 16), lambda i: (i, 0)),
        ],
        out_shape=[
            jax.ShapeDtypeStruct((NPAD, totw), jnp.float32),
            jax.ShapeDtypeStruct((NPAD, 16), jnp.float32),
        ],
    )


@functools.cache
def _sc_layer(h_heads, c_dim):
    hc = h_heads * c_dim
    totw = hc + 16
    nv = c_dim // LANES

    mesh = plsc.VectorSubcoreMesh(core_axis_name="c", subcore_axis_name="s")

    @functools.partial(
        pl.kernel,
        out_type=jax.ShapeDtypeStruct((NPAD, totw), jnp.float32),
        mesh=mesh,
        compiler_params=pltpu.CompilerParams(needs_layout_passes=False,
                                             use_tc_tiling_on_sc=False),
        scratch_types=[
            pltpu.VMEM((S + 16,), jnp.int32),
            pltpu.VMEM((S + 16,), jnp.int32),
            pltpu.VMEM((S + 64,), jnp.int32),
            pltpu.VMEM((16, totw), jnp.float32),
            pltpu.VMEM((16, totw), jnp.float32),
            pltpu.VMEM((16, 16), jnp.float32),
            pltpu.VMEM((hc,), jnp.float32),
            pltpu.VMEM_SHARED((K, totw), jnp.float32),
            pltpu.VMEM_SHARED((K, 16), jnp.float32),
            pltpu.SemaphoreType.DMA,
            pltpu.SemaphoreType.DMA,
            pltpu.SemaphoreType.DMA,
            pltpu.SemaphoreType.DMA,
        ],
    )
    def sc_kernel(hfull_hbm, aux_hbm, src_hbm, dst_hbm, bias_hbm, out_hbm,
                  srcbuf, dstbuf, eidbuf, hb0, hb1, ab0,
                  bbuf, acc_sh, auxsp, sh0, sh1, sa0, ss):
        c = lax.axis_index("c")
        s = lax.axis_index("s")
        tile_e0 = s * EPT
        iota16 = lax.iota(jnp.int32, LANES)
        zv = jnp.zeros((LANES,), jnp.float32)
        sentv = jnp.full((LANES,), SENT, jnp.int32)
        rowv = [(iota16 // h_heads) + (LANES // h_heads) * k for k in range(h_heads)]
        colv = iota16 % h_heads

        pltpu.sync_copy(bias_hbm, bbuf)
        def zid(i, _):
            eidbuf[pl.ds(i * 16, 16)] = sentv
            return 0
        lax.fori_loop(0, (S + 64) // 16, zid, 0)

        def chunk_body(ci, _):
            base = (NSC * ci + c) * K

            def zrow(i, _):
                for v in range(totw // LANES):
                    hb1[i, pl.ds(v * LANES, LANES)] = zv
                return 0
            lax.fori_loop(0, 16, zrow, 0)

            def zcp(z, _):
                pltpu.sync_copy(hb1, acc_sh.at[pl.ds(s * RPT + z * 16, 16)])
                return 0
            lax.fori_loop(0, RPT // 16, zcp, 0)
            pltpu.sync_copy(aux_hbm.at[pl.ds(base + s * RPT, RPT)],
                            auxsp.at[pl.ds(s * RPT, RPT)])
            plsc.subcore_barrier()

            def seg_body(seg, _):
                e0 = tile_e0 + seg * S
                pltpu.sync_copy(src_hbm.at[pl.ds(e0, S)], srcbuf.at[pl.ds(0, S)])
                pltpu.sync_copy(dst_hbm.at[pl.ds(e0, S)], dstbuf.at[pl.ds(0, S)])
                srcbuf[pl.ds(SENT, 16)] = jnp.zeros((16,), jnp.int32)
                dstbuf[pl.ds(SENT, 16)] = jnp.broadcast_to(base, (16,))

                def cbody(t, off):
                    dvec = dstbuf[pl.ds(t * 16, 16)]
                    m = (dvec >= base) & (dvec < base + K)
                    keys = jnp.where(m, 0, 1).astype(jnp.int32)
                    _, vs = plsc.sort_key_val(keys, t * 16 + iota16)
                    eidbuf[pl.ds(off, 16)] = vs
                    return off + jnp.sum(m.astype(jnp.int32))

                n_act = lax.fori_loop(0, S // 16, cbody, jnp.int32(0))
                nblk = (n_act + 15) // 16
                nblk2 = ((nblk + 1) // 2) * 2

                @pl.when(n_act > 0)
                def _pipeline():
                    lb = (n_act // 16) * 16
                    v_last = eidbuf[pl.ds(lb, 16)]
                    eidbuf[pl.ds(lb + 16, 16)] = sentv
                    eidbuf[pl.ds(lb + 32, 16)] = sentv
                    keep = iota16 < (n_act - lb)
                    eidbuf[pl.ds(lb, 16)] = jnp.where(keep, v_last, SENT)

                    hbs = (hb0, hb1)
                    shsem = (sh0, sh1)

                    def issue_h(j, hb, sh):
                        ids = eidbuf[pl.ds(j * 16, 16)]
                        src16 = plsc.load_gather(srcbuf, [ids])
                        pltpu.async_copy(hfull_hbm.at[src16], hb, sh)

                    issue_h(0, hb0, sh0)

                    def qbody(q, _):
                        for pp in range(2):
                            j = q * 2 + pp
                            hb = hbs[pp]
                            pltpu.make_async_copy(
                                hfull_hbm.at[pl.ds(0, 16)], hb, shsem[pp]).wait()

                            @pl.when(j > 0)
                            def _wait_prev_scatter():
                                pltpu.make_async_copy(
                                    hb, acc_sh.at[pl.ds(0, 16)], ss).wait()
                            issue_h(j + 1, hbs[1 - pp], shsem[1 - pp])

                            ids = eidbuf[pl.ds(j * 16, 16)]
                            dst16 = plsc.load_gather(dstbuf, [ids])
                            rel16 = dst16 - base
                            pltpu.async_copy(auxsp.at[rel16], ab0, sa0).wait()

                            for k in range(h_heads):
                                a_s = plsc.load_gather(hb, [rowv[k], hc + colv])
                                a_d = plsc.load_gather(ab0, [rowv[k], 8 + colv])
                                e = a_s + a_d
                                e = jnp.where(e > 0, e, 0.2 * e)
                                ev = jnp.exp(e)
                                eidl = plsc.load_gather(eidbuf, [j * 16 + rowv[k]])
                                ev = jnp.where(eidl != SENT, ev, 0.0)
                                plsc.store_scatter(hb, [rowv[k], hc + colv], ev)

                            def srow(i, _):
                                mv = hb[i, pl.ds(hc, LANES)]
                                for hh in range(h_heads):
                                    m = mv[hh]
                                    for v in range(nv):
                                        sl = pl.ds(hh * c_dim + v * LANES, LANES)
                                        hb[i, sl] = hb[i, sl] * m
                                return 0
                            lax.fori_loop(0, 16, srow, 0)
                            pltpu.async_copy(hb, acc_sh.at[rel16], ss, add=True)
                        return 0

                    lax.fori_loop(0, nblk2 // 2, qbody, 0)
                    pltpu.make_async_copy(
                        hb0, acc_sh.at[pl.ds(0, 16)], ss).wait()
                    pltpu.make_async_copy(
                        hfull_hbm.at[pl.ds(0, 16)],
                        hbs[0] if True else hbs[1],
                        shsem[0]).wait()
                return 0

            lax.fori_loop(0, NSEG, seg_body, 0)
            plsc.subcore_barrier()

            def fbody(t, _):
                r0 = s * RPT + t * 16
                pltpu.sync_copy(acc_sh.at[pl.ds(r0, 16)], hb1)

                def frow(i, _):
                    dv = hb1[i, pl.ds(hc, LANES)]
                    for hh in range(h_heads):
                        d = dv[hh] + 1e-16
                        for v in range(nv):
                            sl = pl.ds(hh * c_dim + v * LANES, LANES)
                            x = hb1[i, sl] / d + bbuf[sl]
                            x = jnp.where(x > 0, x, jnp.exp(x) - 1.0)
                            hb1[i, sl] = x
                    return 0

                lax.fori_loop(0, 16, frow, 0)
                pltpu.sync_copy(hb1, out_hbm.at[pl.ds(base + r0, 16)])
                return 0

            lax.fori_loop(0, RPT // 16, fbody, 0)
            plsc.subcore_barrier()
            return 0

        lax.fori_loop(0, NCHUNK // NSC, chunk_body, 0)

    return sc_kernel


def kernel(x, edge_index, W1, a_src1, a_dst1, b1, W2, a_src2, a_dst2, b2,
           W3, a_src3, a_dst3, b3):
    src = edge_index[0].astype(jnp.int32)
    dst = edge_index[1].astype(jnp.int32)

    def fold(W, a_src, a_dst, heads, cd, fin_pad):
        fin = W.shape[0]
        hc = heads * cd
        was = jnp.sum(W.reshape(fin, heads, cd) * a_src[None], axis=-1)
        wad = jnp.sum(W.reshape(fin, heads, cd) * a_dst[None], axis=-1)
        pad_h = jnp.zeros((fin, 4 - heads), jnp.float32) if heads < 4 else None
        def p4(m):
            return m if pad_h is None else jnp.concatenate([m, pad_h], axis=1)
        zero4 = jnp.zeros((fin, 4), jnp.float32)
        waux = jnp.concatenate([p4(was), zero4, p4(wad), zero4], axis=1)
        waug = jnp.concatenate([W, waux], axis=1)
        if fin < fin_pad:
            waug = jnp.concatenate(
                [waug, jnp.zeros((fin_pad - fin, hc + 16), jnp.float32)], axis=0)
            waux = jnp.concatenate(
                [waux, jnp.zeros((fin_pad - fin, 16), jnp.float32)], axis=0)
        return waug, waux

    xp = jnp.concatenate([x, jnp.zeros((NPAD - N, 64), jnp.float32)], axis=0)

    waug1, waux1 = fold(W1, a_src1, a_dst1, 4, 64, 64)
    h1, aux1 = _dense(64, 272)(xp, waug1, waux1)
    o1 = _sc_layer(4, 64)(h1, aux1, src, dst, b1)
    waug2, waux2 = fold(W2, a_src2, a_dst2, 4, 64, 272)
    h2, aux2 = _dense(272, 272)(o1, waug2, waux2)
    o2 = _sc_layer(4, 64)(h2, aux2, src, dst, b2)
    waug3, waux3 = fold(W3, a_src3, a_dst3, 1, 64, 272)
    h3, aux3 = _dense(272, 80)(o2, waug3, waux3)
    o3 = _sc_layer(1, 64)(h3, aux3, src, dst, b3)
    return o3[:N, :64]


# speedup vs baseline: 1.5407x; 1.5407x over previous
import functools

import jax
import jax.numpy as jnp
from jax import lax
from jax.experimental import pallas as pl
from jax.experimental.pallas import tpu as pltpu
from jax.experimental.pallas import tpu_sc as plsc

N = 50000
E = 800000
NPAD = 51200
K = 6400
NCHUNK = NPAD // K
NSC = 2
NTILE = 16
EPT = E // NTILE
S = 2000
NSEG = EPT // S
SENT = S
RPT = K // NTILE
BR = 512
LANES = 16


def _dense(fin, totw):
    def body(x_ref, w_ref, wa_ref, h_ref, aux_ref):
        xb = x_ref[...]
        h_ref[...] = jnp.dot(xb, w_ref[...], preferred_element_type=jnp.float32)
        aux_ref[...] = jnp.dot(xb, wa_ref[...], preferred_element_type=jnp.float32)

    grid = NPAD // BR
    return pl.pallas_call(
        body,
        grid=(grid,),
        in_specs=[
            pl.BlockSpec((BR, fin), lambda i: (i, 0)),
            pl.BlockSpec((fin, totw), lambda i: (0, 0)),
            pl.BlockSpec((fin, 16), lambda i: (0, 0)),
        ],
        out_specs=[
            pl.BlockSpec((BR, totw), lambda i: (i, 0)),
            pl.BlockSpec((BR, 16), lambda i: (i, 0)),
        ],
        out_shape=[
            jax.ShapeDtypeStruct((NPAD, totw), jnp.float32),
            jax.ShapeDtypeStruct((NPAD, 16), jnp.float32),
        ],
    )


@functools.cache
def _sc_layer(h_heads, c_dim):
    hc = h_heads * c_dim
    totw = hc + 16
    nv = c_dim // LANES

    mesh = plsc.VectorSubcoreMesh(core_axis_name="c", subcore_axis_name="s")

    @functools.partial(
        pl.kernel,
        out_type=jax.ShapeDtypeStruct((NPAD, totw), jnp.float32),
        mesh=mesh,
        compiler_params=pltpu.CompilerParams(needs_layout_passes=False,
                                             use_tc_tiling_on_sc=False),
        scratch_types=[
            pltpu.VMEM((S + 16,), jnp.int32),
            pltpu.VMEM((S + 16,), jnp.int32),
            pltpu.VMEM((S + 64,), jnp.int32),
            pltpu.VMEM((16, totw), jnp.float32),
            pltpu.VMEM((16, 16), jnp.float32),
            pltpu.VMEM((16, totw), jnp.float32),
            pltpu.VMEM((hc,), jnp.float32),
            pltpu.VMEM_SHARED((K, totw), jnp.float32),
            pltpu.VMEM_SHARED((K, 16), jnp.float32),
            pltpu.SemaphoreType.DMA,
            pltpu.SemaphoreType.DMA,
            pltpu.SemaphoreType.DMA,
        ],
    )
    def sc_kernel(hfull_hbm, aux_hbm, src_hbm, dst_hbm, bias_hbm, out_hbm,
                  srcbuf, dstbuf, eidbuf, hb0, ab0,
                  fbuf, bbuf, acc_sh, auxsp, sh0, sa0, ss):
        c = lax.axis_index("c")
        s = lax.axis_index("s")
        tile_e0 = s * EPT
        iota16 = lax.iota(jnp.int32, LANES)
        zv = jnp.zeros((LANES,), jnp.float32)
        sentv = jnp.full((LANES,), SENT, jnp.int32)
        rowv = [(iota16 // h_heads) + (LANES // h_heads) * k for k in range(h_heads)]
        colv = iota16 % h_heads

        pltpu.sync_copy(bias_hbm, bbuf)
        def zid(i, _):
            eidbuf[pl.ds(i * 16, 16)] = sentv
            return 0
        lax.fori_loop(0, (S + 64) // 16, zid, 0)

        def chunk_body(ci, _):
            base = (NSC * ci + c) * K

            def zrow(i, _):
                for v in range(totw // LANES):
                    fbuf[i, pl.ds(v * LANES, LANES)] = zv
                return 0
            lax.fori_loop(0, 16, zrow, 0)

            def zcp(z, _):
                pltpu.sync_copy(fbuf, acc_sh.at[pl.ds(s * RPT + z * 16, 16)])
                return 0
            lax.fori_loop(0, RPT // 16, zcp, 0)
            pltpu.sync_copy(aux_hbm.at[pl.ds(base + s * RPT, RPT)],
                            auxsp.at[pl.ds(s * RPT, RPT)])
            plsc.subcore_barrier()

            def seg_body(seg, _):
                e0 = tile_e0 + seg * S
                pltpu.sync_copy(src_hbm.at[pl.ds(e0, S)], srcbuf.at[pl.ds(0, S)])
                pltpu.sync_copy(dst_hbm.at[pl.ds(e0, S)], dstbuf.at[pl.ds(0, S)])
                srcbuf[pl.ds(SENT, 16)] = jnp.zeros((16,), jnp.int32)
                dstbuf[pl.ds(SENT, 16)] = jnp.broadcast_to(base, (16,))

                def cbody(t, off):
                    dvec = dstbuf[pl.ds(t * 16, 16)]
                    m = (dvec >= base) & (dvec < base + K)
                    keys = jnp.where(m, 0, 1).astype(jnp.int32)
                    _, vs = plsc.sort_key_val(keys, t * 16 + iota16)
                    eidbuf[pl.ds(off, 16)] = vs
                    return off + jnp.sum(m.astype(jnp.int32))

                n_act = lax.fori_loop(0, S // 16, cbody, jnp.int32(0))
                nblk = (n_act + 15) // 16

                @pl.when(n_act > 0)
                def _fixup():
                    last = (nblk - 1) * 16
                    v = eidbuf[pl.ds(last, 16)]
                    keep = iota16 < (n_act - last)
                    eidbuf[pl.ds(last, 16)] = jnp.where(keep, v, SENT)

                def sbody(j, _):
                    ids = eidbuf[pl.ds(j * 16, 16)]
                    src16 = plsc.load_gather(srcbuf, [ids])
                    dst16 = plsc.load_gather(dstbuf, [ids])
                    rel16 = dst16 - base

                    @pl.when(j > 0)
                    def _wait_prev_scatter():
                        pltpu.make_async_copy(
                            hb0, acc_sh.at[pl.ds(0, 16)], ss).wait()

                    cph = pltpu.async_copy(hfull_hbm.at[src16], hb0, sh0)
                    cpb = pltpu.async_copy(auxsp.at[rel16], ab0, sa0)
                    cpb.wait()
                    cph.wait()
                    for k in range(h_heads):
                        a_s = plsc.load_gather(hb0, [rowv[k], hc + colv])
                        a_d = plsc.load_gather(ab0, [rowv[k], 8 + colv])
                        e = a_s + a_d
                        e = jnp.where(e > 0, e, 0.2 * e)
                        ev = jnp.exp(e)
                        eidl = plsc.load_gather(eidbuf, [j * 16 + rowv[k]])
                        ev = jnp.where(eidl != SENT, ev, 0.0)
                        plsc.store_scatter(hb0, [rowv[k], hc + colv], ev)

                    def srow(i, _):
                        mv = hb0[i, pl.ds(hc, LANES)]
                        for hh in range(h_heads):
                            m = mv[hh]
                            for v in range(nv):
                                sl = pl.ds(hh * c_dim + v * LANES, LANES)
                                hb0[i, sl] = hb0[i, sl] * m
                        return 0
                    lax.fori_loop(0, 16, srow, 0)
                    pltpu.async_copy(hb0, acc_sh.at[rel16], ss, add=True)
                    return 0

                lax.fori_loop(0, nblk, sbody, 0)

                @pl.when(nblk > 0)
                def _drain_scatter():
                    pltpu.make_async_copy(hb0, acc_sh.at[pl.ds(0, 16)], ss).wait()
                return 0

            lax.fori_loop(0, NSEG, seg_body, 0)
            plsc.subcore_barrier()

            def fbody(t, _):
                r0 = s * RPT + t * 16
                pltpu.sync_copy(acc_sh.at[pl.ds(r0, 16)], fbuf)

                def frow(i, _):
                    dv = fbuf[i, pl.ds(hc, LANES)]
                    for hh in range(h_heads):
                        d = dv[hh] + 1e-16
                        for v in range(nv):
                            sl = pl.ds(hh * c_dim + v * LANES, LANES)
                            x = fbuf[i, sl] / d + bbuf[sl]
                            x = jnp.where(x > 0, x, jnp.exp(x) - 1.0)
                            fbuf[i, sl] = x
                    return 0

                lax.fori_loop(0, 16, frow, 0)
                pltpu.sync_copy(fbuf, out_hbm.at[pl.ds(base + r0, 16)])
                return 0

            lax.fori_loop(0, RPT // 16, fbody, 0)
            plsc.subcore_barrier()
            return 0

        lax.fori_loop(0, NCHUNK // NSC, chunk_body, 0)

    return sc_kernel


def kernel(x, edge_index, W1, a_src1, a_dst1, b1, W2, a_src2, a_dst2, b2,
           W3, a_src3, a_dst3, b3):
    src = edge_index[0].astype(jnp.int32)
    dst = edge_index[1].astype(jnp.int32)

    def fold(W, a_src, a_dst, heads, cd, fin_pad):
        fin = W.shape[0]
        hc = heads * cd
        was = jnp.sum(W.reshape(fin, heads, cd) * a_src[None], axis=-1)
        wad = jnp.sum(W.reshape(fin, heads, cd) * a_dst[None], axis=-1)
        pad_h = jnp.zeros((fin, 4 - heads), jnp.float32) if heads < 4 else None
        def p4(m):
            return m if pad_h is None else jnp.concatenate([m, pad_h], axis=1)
        zero4 = jnp.zeros((fin, 4), jnp.float32)
        waux = jnp.concatenate([p4(was), zero4, p4(wad), zero4], axis=1)
        waug = jnp.concatenate([W, waux], axis=1)
        if fin < fin_pad:
            waug = jnp.concatenate(
                [waug, jnp.zeros((fin_pad - fin, hc + 16), jnp.float32)], axis=0)
            waux = jnp.concatenate(
                [waux, jnp.zeros((fin_pad - fin, 16), jnp.float32)], axis=0)
        return waug, waux

    xp = jnp.concatenate([x, jnp.zeros((NPAD - N, 64), jnp.float32)], axis=0)

    waug1, waux1 = fold(W1, a_src1, a_dst1, 4, 64, 64)
    h1, aux1 = _dense(64, 272)(xp, waug1, waux1)
    o1 = _sc_layer(4, 64)(h1, aux1, src, dst, b1)
    waug2, waux2 = fold(W2, a_src2, a_dst2, 4, 64, 272)
    h2, aux2 = _dense(272, 272)(o1, waug2, waux2)
    o2 = _sc_layer(4, 64)(h2, aux2, src, dst, b2)
    waug3, waux3 = fold(W3, a_src3, a_dst3, 1, 64, 272)
    h3, aux3 = _dense(272, 80)(o2, waug3, waux3)
    o3 = _sc_layer(1, 64)(h3, aux3, src, dst, b3)
    return o3[:N, :64]
